# Initial kernel scaffold; baseline (speedup 1.0000x reference)
import jax
import jax.numpy as jnp
from jax.experimental import pallas as pl


def kernel(x, edge_index, Wl1, bl1, Wr1, Wl2, bl2, Wr2, g1, b1, g2, b2, Wp, bp):
    raise NotImplementedError("write your pallas kernel here")


# speedup vs baseline: 2.9561x; 2.9561x over previous
import functools

import jax
import jax.numpy as jnp
from jax import lax
from jax.experimental import pallas as pl
from jax.experimental.pallas import tpu as pltpu, tpu_sc as plsc

N = 10000
E = 320000
D = 128
H = 256
O = 128

NPAD = 10112
SINK = N
E_PAD = 327680
NBLK = E_PAD // 128
HI = lax.Precision.HIGHEST

_f32 = jnp.float32
_i32 = jnp.int32


def _zero_fill(buf, nrows):
  zero16 = jnp.zeros((16,), _f32)
  def zrow(r, _):
    for k in range(8):
      buf[r, pl.ds(k * 16, 16)] = zero16
    return 0
  lax.fori_loop(0, nrows, zrow, 0)


def _zero_shared(shared_ref, zsrc, zb, nrows_chunk):
  for q in range(512 // nrows_chunk):
    pltpu.sync_copy(zsrc, shared_ref.at[pl.ds(zb + q * nrows_chunk,
                                              nrows_chunk)])
  pltpu.sync_copy(zsrc.at[pl.ds(0, 120)], shared_ref.at[pl.ds(zb + 512, 120)])


def _stage_out(shared_ref, hbm_ref, stage_ref, ch, src0, dst0, n):
  for q in range(n // ch):
    pltpu.sync_copy(shared_ref.at[pl.ds(src0 + q * ch, ch)], stage_ref)
    pltpu.sync_copy(stage_ref, hbm_ref.at[pl.ds(dst0 + q * ch, ch)])
  r = n % ch
  if r:
    q = (n // ch) * ch
    pltpu.sync_copy(shared_ref.at[pl.ds(src0 + q, r)],
                    stage_ref.at[pl.ds(0, r)])
    pltpu.sync_copy(stage_ref.at[pl.ds(0, r)],
                    hbm_ref.at[pl.ds(dst0 + q, r)])


def _copy_out(shared_ref, hbm_ref, stage_ref, ch, c, s):
  ob = s * 624
  _stage_out(shared_ref, hbm_ref, stage_ref, ch, ob, c * N + ob, 624)
  @pl.when(s == 0)
  def _():
    _stage_out(shared_ref, hbm_ref, stage_ref, ch, 9984, c * N + 9984, 16)


_MESH = dict(core_axis_name="c", subcore_axis_name="s",
             num_cores=2, num_subcores=16)
IB = 8


@functools.lru_cache(maxsize=None)
def _make_sc_agg(edge_split):
  K = 1 if edge_split else 2
  BT = (NBLK // 32) if edge_split else (NBLK // 16)

  def body(t_all, src2, dst2, out_all, acc, srcv, dstv, rows, sem):
    c = lax.axis_index("c")
    s = lax.axis_index("s")

    _zero_fill(rows, K * 128)
    _zero_shared(acc, rows, s * 632, K * 128)
    plsc.subcore_barrier()

    if edge_split:
      blk0 = (c * 16 + s) * BT
    else:
      blk0 = s * BT
      off = jnp.full((16,), c * N, _i32)

    def chunk(t, _):
      rb = blk0 + t * IB
      pltpu.sync_copy(src2.at[pl.ds(rb, IB)], srcv)
      pltpu.sync_copy(dst2.at[pl.ds(rb, IB)], dstv)
      if not edge_split:
        for j in range(IB):
          for k in range(8):
            srcv[j, pl.ds(k * 16, 16)] = srcv[j, pl.ds(k * 16, 16)] + off
      for u in range(IB // K):
        hs = [
            pltpu.async_copy(t_all.at[srcv.at[u * K + j]],
                             rows.at[pl.ds(j * 128, 128)], sem)
            for j in range(K)
        ]
        for h_ in hs:
          h_.wait()
        for j in range(K):
          pltpu.sync_copy(rows.at[pl.ds(j * 128, 128)],
                          acc.at[dstv.at[u * K + j]], add=True)
      return 0

    lax.fori_loop(0, BT // IB, chunk, 0)
    plsc.subcore_barrier()
    _copy_out(acc, out_all, rows, K * 128, c, s)

  return pl.kernel(
      body,
      out_type=jax.ShapeDtypeStruct((2 * N, 128), _f32),
      mesh=plsc.VectorSubcoreMesh(**_MESH),
      scratch_types=[
          pltpu.VMEM_SHARED((NPAD, 128), _f32),
          pltpu.VMEM((IB, 128), _i32),
          pltpu.VMEM((IB, 128), _i32),
          pltpu.VMEM((K * 128, 128), _f32),
          pltpu.SemaphoreType.DMA,
      ])


@functools.lru_cache(maxsize=None)
def _make_sc_deg():
  BT = NBLK // 32

  def body(dst2, deg_out, dega, dstv, onesb, sem):
    c = lax.axis_index("c")
    s = lax.axis_index("s")

    _zero_fill(onesb, 128)
    _zero_shared(dega, onesb, s * 632, 128)
    one16 = jnp.ones((16,), _f32)
    def orow(r, _):
      for k in range(8):
        onesb[r, pl.ds(k * 16, 16)] = one16
      return 0
    lax.fori_loop(0, 128, orow, 0)
    plsc.subcore_barrier()

    blk0 = (c * 16 + s) * BT

    def chunk(t, _):
      rb = blk0 + t * IB
      pltpu.sync_copy(dst2.at[pl.ds(rb, IB)], dstv)
      for u in range(IB):
        pltpu.sync_copy(onesb, dega.at[dstv.at[u]], add=True)
      return 0

    lax.fori_loop(0, BT // IB, chunk, 0)
    plsc.subcore_barrier()
    _copy_out(dega, deg_out, onesb, 128, c, s)

  return pl.kernel(
      body,
      out_type=jax.ShapeDtypeStruct((2 * N, 128), _f32),
      mesh=plsc.VectorSubcoreMesh(**_MESH),
      scratch_types=[
          pltpu.VMEM_SHARED((NPAD, 128), _f32),
          pltpu.VMEM((IB, 128), _i32),
          pltpu.VMEM((128, 128), _f32),
          pltpu.SemaphoreType.DMA,
      ])


NB = 10
RB = N // NB


def _deg_inv(deg_ref):
  d = deg_ref[0][:, 0:1] + deg_ref[1][:, 0:1]
  return 1.0 / jnp.maximum(d, 1.0)


def _bn_stats(i, h, hbuf, s1, s2):
  hbuf[pl.ds(i * RB, RB), :] = h

  @pl.when(i == 0)
  def _():
    s1[...] = jnp.zeros_like(s1)
    s2[...] = jnp.zeros_like(s2)
  s1[...] += jnp.sum(h, axis=0, keepdims=True)
  s2[...] += jnp.sum(h * h, axis=0, keepdims=True)


def _bn_apply(i, hbuf, s1, s2, g_ref, b_ref):
  mu = s1[...] / N
  var = s2[...] / N - mu * mu
  hb = hbuf[pl.ds(i * RB, RB), :]
  return (hb - mu) * lax.rsqrt(var + 1e-5) * g_ref[...] + b_ref[...]


def _tc1_body(agg_ref, deg_ref, x_ref, wl_ref, bl_ref, wr_ref, g_ref, b_ref,
              out_ref, hbuf, s1, s2):
  p = pl.program_id(0)
  i = pl.program_id(1)

  @pl.when(p == 0)
  def _():
    m = (agg_ref[0] + agg_ref[1]) * _deg_inv(deg_ref)
    h = (jnp.dot(m, wl_ref[...], precision=HI)
         + jnp.dot(x_ref[...], wr_ref[...], precision=HI) + bl_ref[...])
    _bn_stats(i, jnp.maximum(h, 0.0), hbuf, s1, s2)

  @pl.when(p == 1)
  def _():
    h = _bn_apply(i, hbuf, s1, s2, g_ref, b_ref)
    out_ref[0] = h[:, 0:128]
    out_ref[1] = h[:, 128:256]


_tc1 = pl.pallas_call(
    _tc1_body,
    grid=(2, NB),
    in_specs=[
        pl.BlockSpec((2, RB, 128), lambda p, i: (0, i, 0)),
        pl.BlockSpec((2, RB, 128), lambda p, i: (0, i, 0)),
        pl.BlockSpec((RB, 128), lambda p, i: (i, 0)),
        pl.BlockSpec((128, H), lambda p, i: (0, 0)),
        pl.BlockSpec((1, H), lambda p, i: (0, 0)),
        pl.BlockSpec((128, H), lambda p, i: (0, 0)),
        pl.BlockSpec((1, H), lambda p, i: (0, 0)),
        pl.BlockSpec((1, H), lambda p, i: (0, 0)),
    ],
    out_specs=pl.BlockSpec((2, RB, 128), lambda p, i: (0, i, 0)),
    out_shape=jax.ShapeDtypeStruct((2, N, 128), _f32),
    scratch_shapes=[pltpu.VMEM((N, H), _f32), pltpu.VMEM((1, H), _f32),
                    pltpu.VMEM((1, H), _f32)])


def _tc2_body(agg_ref, deg_ref, h1_ref, wl_ref, bl_ref, wr_ref, g_ref, b_ref,
              wp_ref, bp_ref, out_ref, hbuf, s1, s2):
  p = pl.program_id(0)
  i = pl.program_id(1)

  @pl.when(p == 0)
  def _():
    inv = _deg_inv(deg_ref)
    wl = wl_ref[...]
    wr = wr_ref[...]
    h = (jnp.dot(agg_ref[0] * inv, wl[0:128, :], precision=HI)
         + jnp.dot(agg_ref[1] * inv, wl[128:256, :], precision=HI)
         + jnp.dot(h1_ref[0], wr[0:128, :], precision=HI)
         + jnp.dot(h1_ref[1], wr[128:256, :], precision=HI)
         + bl_ref[...])
    _bn_stats(i, jnp.maximum(h, 0.0), hbuf, s1, s2)

  @pl.when(p == 1)
  def _():
    h = _bn_apply(i, hbuf, s1, s2, g_ref, b_ref)
    out_ref[...] = jnp.maximum(jnp.dot(h, wp_ref[...], precision=HI)
                               + bp_ref[...], 0.0)


_tc2 = pl.pallas_call(
    _tc2_body,
    grid=(2, NB),
    in_specs=[
        pl.BlockSpec((2, RB, 128), lambda p, i: (0, i, 0)),
        pl.BlockSpec((2, RB, 128), lambda p, i: (0, i, 0)),
        pl.BlockSpec((2, RB, 128), lambda p, i: (0, i, 0)),
        pl.BlockSpec((H, H), lambda p, i: (0, 0)),
        pl.BlockSpec((1, H), lambda p, i: (0, 0)),
        pl.BlockSpec((H, H), lambda p, i: (0, 0)),
        pl.BlockSpec((1, H), lambda p, i: (0, 0)),
        pl.BlockSpec((1, H), lambda p, i: (0, 0)),
        pl.BlockSpec((H, O), lambda p, i: (0, 0)),
        pl.BlockSpec((1, O), lambda p, i: (0, 0)),
    ],
    out_specs=pl.BlockSpec((RB, O), lambda p, i: (i, 0)),
    out_shape=jax.ShapeDtypeStruct((N, O), _f32),
    scratch_shapes=[pltpu.VMEM((N, H), _f32), pltpu.VMEM((1, H), _f32),
                    pltpu.VMEM((1, H), _f32)])


def kernel(x, edge_index, Wl1, bl1, Wr1, Wl2, bl2, Wr2, g1, b1, g2, b2, Wp,
           bp):
  src = edge_index[0]
  dst = edge_index[1]
  pad = E_PAD - E
  src_p = jnp.concatenate([src, jnp.zeros((pad,), _i32)]).reshape(-1, 128)
  dst_p = jnp.concatenate([dst, jnp.full((pad,), SINK, _i32)]).reshape(-1, 128)

  agg1_all = _make_sc_agg(True)(x, src_p, dst_p)
  degw = _make_sc_deg()(dst_p).reshape(2, N, 128)
  h13 = _tc1(agg1_all.reshape(2, N, 128), degw, x, Wl1,
             bl1.reshape(1, -1), Wr1, g1.reshape(1, -1), b1.reshape(1, -1))
  h1_all = h13.reshape(2 * N, 128)
  agg2_all = _make_sc_agg(False)(h1_all, src_p, dst_p)
  z = _tc2(agg2_all.reshape(2, N, 128), degw, h13, Wl2,
           bl2.reshape(1, -1), Wr2, g2.reshape(1, -1), b2.reshape(1, -1),
           Wp, bp.reshape(1, -1))
  return z


# speedup vs baseline: 2.9601x; 1.0014x over previous
import functools

import jax
import jax.numpy as jnp
from jax import lax
from jax.experimental import pallas as pl
from jax.experimental.pallas import tpu as pltpu, tpu_sc as plsc

N = 10000
E = 320000
D = 128
H = 256
O = 128

NPAD = 10112
SINK = N
E_PAD = 327680
NBLK = E_PAD // 128
HI = lax.Precision.HIGHEST

_f32 = jnp.float32
_i32 = jnp.int32


def _zero_fill(buf, nrows):
  zero16 = jnp.zeros((16,), _f32)
  def zrow(r, _):
    for k in range(8):
      buf[r, pl.ds(k * 16, 16)] = zero16
    return 0
  lax.fori_loop(0, nrows, zrow, 0)


def _zero_shared(shared_ref, zsrc, zb, nrows_chunk):
  for q in range(512 // nrows_chunk):
    pltpu.sync_copy(zsrc, shared_ref.at[pl.ds(zb + q * nrows_chunk,
                                              nrows_chunk)])
  pltpu.sync_copy(zsrc.at[pl.ds(0, 120)], shared_ref.at[pl.ds(zb + 512, 120)])


def _stage_out(shared_ref, hbm_ref, stage_ref, ch, src0, dst0, n):
  for q in range(n // ch):
    pltpu.sync_copy(shared_ref.at[pl.ds(src0 + q * ch, ch)], stage_ref)
    pltpu.sync_copy(stage_ref, hbm_ref.at[pl.ds(dst0 + q * ch, ch)])
  r = n % ch
  if r:
    q = (n // ch) * ch
    pltpu.sync_copy(shared_ref.at[pl.ds(src0 + q, r)],
                    stage_ref.at[pl.ds(0, r)])
    pltpu.sync_copy(stage_ref.at[pl.ds(0, r)],
                    hbm_ref.at[pl.ds(dst0 + q, r)])


def _copy_out(shared_ref, hbm_ref, stage_ref, ch, c, s):
  ob = s * 624
  _stage_out(shared_ref, hbm_ref, stage_ref, ch, ob, c * N + ob, 624)
  @pl.when(s == 0)
  def _():
    _stage_out(shared_ref, hbm_ref, stage_ref, ch, 9984, c * N + 9984, 16)


_MESH = dict(core_axis_name="c", subcore_axis_name="s",
             num_cores=2, num_subcores=16)
IB = 8


@functools.lru_cache(maxsize=None)
def _make_sc_agg(edge_split):
  K = 1 if edge_split else 2
  BT = (NBLK // 32) if edge_split else (NBLK // 16)

  def body(t_all, src2, dst2, out_all, acc, srcv, dstv, rows, sem):
    c = lax.axis_index("c")
    s = lax.axis_index("s")

    _zero_fill(rows, K * 128)
    _zero_shared(acc, rows, s * 632, K * 128)
    plsc.subcore_barrier()

    if edge_split:
      blk0 = (c * 16 + s) * BT
    else:
      blk0 = s * BT
      off = jnp.full((16,), c * N, _i32)

    def chunk(t, _):
      rb = blk0 + t * IB
      pltpu.sync_copy(src2.at[pl.ds(rb, IB)], srcv)
      pltpu.sync_copy(dst2.at[pl.ds(rb, IB)], dstv)
      if not edge_split:
        for j in range(IB):
          for k in range(8):
            srcv[j, pl.ds(k * 16, 16)] = srcv[j, pl.ds(k * 16, 16)] + off
      for u in range(IB // K):
        hs = [
            pltpu.async_copy(t_all.at[srcv.at[u * K + j]],
                             rows.at[pl.ds(j * 128, 128)], sem)
            for j in range(K)
        ]
        for h_ in hs:
          h_.wait()
        for j in range(K):
          pltpu.sync_copy(rows.at[pl.ds(j * 128, 128)],
                          acc.at[dstv.at[u * K + j]], add=True)
      return 0

    lax.fori_loop(0, BT // IB, chunk, 0)
    plsc.subcore_barrier()
    _copy_out(acc, out_all, rows, K * 128, c, s)

  return pl.kernel(
      body,
      out_type=jax.ShapeDtypeStruct((2 * N, 128), _f32),
      mesh=plsc.VectorSubcoreMesh(**_MESH),
      scratch_types=[
          pltpu.VMEM_SHARED((NPAD, 128), _f32),
          pltpu.VMEM((IB, 128), _i32),
          pltpu.VMEM((IB, 128), _i32),
          pltpu.VMEM((K * 128, 128), _f32),
          pltpu.SemaphoreType.DMA,
      ])


@functools.lru_cache(maxsize=None)
def _make_sc_deg():
  BT = NBLK // 32

  def body(dst2, deg_out, dega, dstv, onesb, sem):
    c = lax.axis_index("c")
    s = lax.axis_index("s")

    _zero_fill(onesb, 128)
    _zero_shared(dega, onesb, s * 632, 128)
    one16 = jnp.ones((16,), _f32)
    def orow(r, _):
      for k in range(8):
        onesb[r, pl.ds(k * 16, 16)] = one16
      return 0
    lax.fori_loop(0, 128, orow, 0)
    plsc.subcore_barrier()

    blk0 = (c * 16 + s) * BT

    def chunk(t, _):
      rb = blk0 + t * IB
      pltpu.sync_copy(dst2.at[pl.ds(rb, IB)], dstv)
      for u in range(IB):
        pltpu.sync_copy(onesb, dega.at[dstv.at[u]], add=True)
      return 0

    lax.fori_loop(0, BT // IB, chunk, 0)
    plsc.subcore_barrier()
    _copy_out(dega, deg_out, onesb, 128, c, s)

  return pl.kernel(
      body,
      out_type=jax.ShapeDtypeStruct((2 * N, 128), _f32),
      mesh=plsc.VectorSubcoreMesh(**_MESH),
      scratch_types=[
          pltpu.VMEM_SHARED((NPAD, 128), _f32),
          pltpu.VMEM((IB, 128), _i32),
          pltpu.VMEM((128, 128), _f32),
          pltpu.SemaphoreType.DMA,
      ])


NB = 10
RB = N // NB


def _deg_inv(deg_ref):
  d = deg_ref[0][:, 0:1] + deg_ref[1][:, 0:1]
  return 1.0 / jnp.maximum(d, 1.0)


def _bn_stats(i, h, hbuf, s1, s2):
  hbuf[pl.ds(i * RB, RB), :] = h

  @pl.when(i == 0)
  def _():
    s1[...] = jnp.zeros_like(s1)
    s2[...] = jnp.zeros_like(s2)
  s1[...] += jnp.sum(h, axis=0, keepdims=True)
  s2[...] += jnp.sum(h * h, axis=0, keepdims=True)


def _bn_apply(i, hbuf, s1, s2, g_ref, b_ref):
  mu = s1[...] / N
  var = s2[...] / N - mu * mu
  hb = hbuf[pl.ds(i * RB, RB), :]
  return (hb - mu) * lax.rsqrt(var + 1e-5) * g_ref[...] + b_ref[...]


def _tc1_body(agg_ref, deg_ref, x_ref, wl_ref, bl_ref, wr_ref, g_ref, b_ref,
              out_ref, hbuf, s1, s2):
  p = pl.program_id(0)
  i = pl.program_id(1)

  @pl.when(p == 0)
  def _():
    m = (agg_ref[0] + agg_ref[1]) * _deg_inv(deg_ref)
    h = (jnp.dot(m, wl_ref[...], precision=HI)
         + jnp.dot(x_ref[...], wr_ref[...], precision=HI) + bl_ref[...])
    _bn_stats(i, jnp.maximum(h, 0.0), hbuf, s1, s2)

  @pl.when(p == 1)
  def _():
    h = _bn_apply(i, hbuf, s1, s2, g_ref, b_ref)
    out_ref[0] = h[:, 0:128]
    out_ref[1] = h[:, 128:256]


_tc1 = pl.pallas_call(
    _tc1_body,
    grid=(2, NB),
    in_specs=[
        pl.BlockSpec((2, RB, 128), lambda p, i: (0, i, 0)),
        pl.BlockSpec((2, RB, 128), lambda p, i: (0, i, 0)),
        pl.BlockSpec((RB, 128), lambda p, i: (i, 0)),
        pl.BlockSpec((128, H), lambda p, i: (0, 0)),
        pl.BlockSpec((1, H), lambda p, i: (0, 0)),
        pl.BlockSpec((128, H), lambda p, i: (0, 0)),
        pl.BlockSpec((1, H), lambda p, i: (0, 0)),
        pl.BlockSpec((1, H), lambda p, i: (0, 0)),
    ],
    out_specs=pl.BlockSpec((2, RB, 128), lambda p, i: (0, i, 0)),
    out_shape=jax.ShapeDtypeStruct((2, N, 128), _f32),
    scratch_shapes=[pltpu.VMEM((N, H), _f32), pltpu.VMEM((1, H), _f32),
                    pltpu.VMEM((1, H), _f32)])


def _tc2_body(agg_ref, deg_ref, h1_ref, wl_ref, bl_ref, wr_ref, g_ref, b_ref,
              wp_ref, bp_ref, out_ref, hbuf, s1, s2):
  p = pl.program_id(0)
  i = pl.program_id(1)

  @pl.when(p == 0)
  def _():
    inv = _deg_inv(deg_ref)
    wl = wl_ref[...]
    wr = wr_ref[...]
    h = (jnp.dot(agg_ref[0] * inv, wl[0:128, :], precision=HI)
         + jnp.dot(agg_ref[1] * inv, wl[128:256, :], precision=HI)
         + jnp.dot(h1_ref[0], wr[0:128, :], precision=HI)
         + jnp.dot(h1_ref[1], wr[128:256, :], precision=HI)
         + bl_ref[...])
    _bn_stats(i, jnp.maximum(h, 0.0), hbuf, s1, s2)

  @pl.when(p == 1)
  def _():
    h = _bn_apply(i, hbuf, s1, s2, g_ref, b_ref)
    out_ref[...] = jnp.maximum(jnp.dot(h, wp_ref[...], precision=HI)
                               + bp_ref[...], 0.0)


_tc2 = pl.pallas_call(
    _tc2_body,
    grid=(2, NB),
    in_specs=[
        pl.BlockSpec((2, RB, 128), lambda p, i: (0, i, 0)),
        pl.BlockSpec((2, RB, 128), lambda p, i: (0, i, 0)),
        pl.BlockSpec((2, RB, 128), lambda p, i: (0, i, 0)),
        pl.BlockSpec((H, H), lambda p, i: (0, 0)),
        pl.BlockSpec((1, H), lambda p, i: (0, 0)),
        pl.BlockSpec((H, H), lambda p, i: (0, 0)),
        pl.BlockSpec((1, H), lambda p, i: (0, 0)),
        pl.BlockSpec((1, H), lambda p, i: (0, 0)),
        pl.BlockSpec((H, O), lambda p, i: (0, 0)),
        pl.BlockSpec((1, O), lambda p, i: (0, 0)),
    ],
    out_specs=pl.BlockSpec((RB, O), lambda p, i: (i, 0)),
    out_shape=jax.ShapeDtypeStruct((N, O), _f32),
    scratch_shapes=[pltpu.VMEM((N, H), _f32), pltpu.VMEM((1, H), _f32),
                    pltpu.VMEM((1, H), _f32)])


def kernel(x, edge_index, Wl1, bl1, Wr1, Wl2, bl2, Wr2, g1, b1, g2, b2, Wp,
           bp):
  src = edge_index[0]
  dst = edge_index[1]
  pad = E_PAD - E
  src_p = jnp.concatenate([src, jnp.zeros((pad,), _i32)]).reshape(-1, 128)
  sink = SINK + (jnp.arange(pad, dtype=_i32) % (NPAD - N))
  dst_p = jnp.concatenate([dst, sink]).reshape(-1, 128)

  agg1_all = _make_sc_agg(True)(x, src_p, dst_p)
  degw = _make_sc_deg()(dst_p).reshape(2, N, 128)
  h13 = _tc1(agg1_all.reshape(2, N, 128), degw, x, Wl1,
             bl1.reshape(1, -1), Wr1, g1.reshape(1, -1), b1.reshape(1, -1))
  h1_all = h13.reshape(2 * N, 128)
  agg2_all = _make_sc_agg(False)(h1_all, src_p, dst_p)
  z = _tc2(agg2_all.reshape(2, N, 128), degw, h13, Wl2,
           bl2.reshape(1, -1), Wr2, g2.reshape(1, -1), b2.reshape(1, -1),
           Wp, bp.reshape(1, -1))
  return z


# speedup vs baseline: 3.1366x; 1.0596x over previous
import functools

import jax
import jax.numpy as jnp
from jax import lax
from jax.experimental import pallas as pl
from jax.experimental.pallas import tpu as pltpu, tpu_sc as plsc

N = 10000
E = 320000
D = 128
H = 256
O = 128

NPAD = 10112
SINK = N
E_PAD = 327680
NBLK = E_PAD // 128
HI = lax.Precision.HIGHEST

_f32 = jnp.float32
_i32 = jnp.int32


def _zero_fill(buf, nrows):
  zero16 = jnp.zeros((16,), _f32)
  def zrow(r, _):
    for k in range(8):
      buf[r, pl.ds(k * 16, 16)] = zero16
    return 0
  lax.fori_loop(0, nrows, zrow, 0)


def _zero_shared(shared_ref, zsrc, zb, nrows_chunk):
  for q in range(512 // nrows_chunk):
    pltpu.sync_copy(zsrc, shared_ref.at[pl.ds(zb + q * nrows_chunk,
                                              nrows_chunk)])
  pltpu.sync_copy(zsrc.at[pl.ds(0, 120)], shared_ref.at[pl.ds(zb + 512, 120)])


def _stage_out(shared_ref, hbm_ref, stage_ref, ch, src0, dst0, n):
  for q in range(n // ch):
    pltpu.sync_copy(shared_ref.at[pl.ds(src0 + q * ch, ch)], stage_ref)
    pltpu.sync_copy(stage_ref, hbm_ref.at[pl.ds(dst0 + q * ch, ch)])
  r = n % ch
  if r:
    q = (n // ch) * ch
    pltpu.sync_copy(shared_ref.at[pl.ds(src0 + q, r)],
                    stage_ref.at[pl.ds(0, r)])
    pltpu.sync_copy(stage_ref.at[pl.ds(0, r)],
                    hbm_ref.at[pl.ds(dst0 + q, r)])


def _copy_out(shared_ref, hbm_ref, stage_ref, ch, c, s):
  ob = s * 624
  _stage_out(shared_ref, hbm_ref, stage_ref, ch, ob, c * N + ob, 624)
  @pl.when(s == 0)
  def _():
    _stage_out(shared_ref, hbm_ref, stage_ref, ch, 9984, c * N + 9984, 16)


_MESH = dict(core_axis_name="c", subcore_axis_name="s",
             num_cores=2, num_subcores=16)
IB = 8


@functools.lru_cache(maxsize=None)
def _make_sc_agg(edge_split):
  K = 2
  BT = (NBLK // 32) if edge_split else (NBLK // 16)

  def body(t_all, src2, dst2, out_all, acc, srcv, dstv, rows, semg, sems):
    c = lax.axis_index("c")
    s = lax.axis_index("s")

    _zero_fill(rows, K * 128)
    _zero_shared(acc, rows, s * 632, K * 128)
    plsc.subcore_barrier()

    if edge_split:
      blk0 = (c * 16 + s) * BT
    else:
      blk0 = s * BT
      off = jnp.full((16,), c * N, _i32)

    def rbuf(u):
      return rows.at[pl.ds((u % K) * 128, 128)]

    def chunk(t, _):
      rb = blk0 + t * IB
      pltpu.sync_copy(src2.at[pl.ds(rb, IB)], srcv)
      pltpu.sync_copy(dst2.at[pl.ds(rb, IB)], dstv)
      if not edge_split:
        for j in range(IB):
          for k in range(8):
            srcv[j, pl.ds(k * 16, 16)] = srcv[j, pl.ds(k * 16, 16)] + off
      g = [None] * IB
      sc = [None] * IB
      g[0] = pltpu.async_copy(t_all.at[srcv.at[0]], rbuf(0), semg)
      for u in range(IB):
        g[u].wait()
        sc[u] = pltpu.async_copy(rbuf(u), acc.at[dstv.at[u]], sems, add=True)
        if u + 1 < IB:
          if u >= 1:
            sc[u - 1].wait()
          g[u + 1] = pltpu.async_copy(t_all.at[srcv.at[u + 1]], rbuf(u + 1),
                                      semg)
      sc[IB - 2].wait()
      sc[IB - 1].wait()
      return 0

    lax.fori_loop(0, BT // IB, chunk, 0)
    plsc.subcore_barrier()
    _copy_out(acc, out_all, rows, K * 128, c, s)

  return pl.kernel(
      body,
      out_type=jax.ShapeDtypeStruct((2 * N, 128), _f32),
      mesh=plsc.VectorSubcoreMesh(**_MESH),
      scratch_types=[
          pltpu.VMEM_SHARED((NPAD, 128), _f32),
          pltpu.VMEM((IB, 128), _i32),
          pltpu.VMEM((IB, 128), _i32),
          pltpu.VMEM((K * 128, 128), _f32),
          pltpu.SemaphoreType.DMA,
          pltpu.SemaphoreType.DMA,
      ])


@functools.lru_cache(maxsize=None)
def _make_sc_deg():
  BT = NBLK // 32

  def body(dst2, deg_out, dega, dstv, onesb, sem):
    c = lax.axis_index("c")
    s = lax.axis_index("s")

    _zero_fill(onesb, 128)
    _zero_shared(dega, onesb, s * 632, 128)
    one16 = jnp.ones((16,), _f32)
    def orow(r, _):
      for k in range(8):
        onesb[r, pl.ds(k * 16, 16)] = one16
      return 0
    lax.fori_loop(0, 128, orow, 0)
    plsc.subcore_barrier()

    blk0 = (c * 16 + s) * BT

    def chunk(t, _):
      rb = blk0 + t * IB
      pltpu.sync_copy(dst2.at[pl.ds(rb, IB)], dstv)
      hs = [pltpu.async_copy(onesb, dega.at[dstv.at[u]], sem, add=True)
            for u in range(IB)]
      for h_ in hs:
        h_.wait()
      return 0

    lax.fori_loop(0, BT // IB, chunk, 0)
    plsc.subcore_barrier()
    _copy_out(dega, deg_out, onesb, 128, c, s)

  return pl.kernel(
      body,
      out_type=jax.ShapeDtypeStruct((2 * N, 128), _f32),
      mesh=plsc.VectorSubcoreMesh(**_MESH),
      scratch_types=[
          pltpu.VMEM_SHARED((NPAD, 128), _f32),
          pltpu.VMEM((IB, 128), _i32),
          pltpu.VMEM((128, 128), _f32),
          pltpu.SemaphoreType.DMA,
      ])


NB = 10
RB = N // NB


def _deg_inv(deg_ref):
  d = deg_ref[0][:, 0:1] + deg_ref[1][:, 0:1]
  return 1.0 / jnp.maximum(d, 1.0)


def _bn_stats(i, h, hbuf, s1, s2):
  hbuf[pl.ds(i * RB, RB), :] = h

  @pl.when(i == 0)
  def _():
    s1[...] = jnp.zeros_like(s1)
    s2[...] = jnp.zeros_like(s2)
  s1[...] += jnp.sum(h, axis=0, keepdims=True)
  s2[...] += jnp.sum(h * h, axis=0, keepdims=True)


def _bn_apply(i, hbuf, s1, s2, g_ref, b_ref):
  mu = s1[...] / N
  var = s2[...] / N - mu * mu
  hb = hbuf[pl.ds(i * RB, RB), :]
  return (hb - mu) * lax.rsqrt(var + 1e-5) * g_ref[...] + b_ref[...]


def _tc1_body(agg_ref, deg_ref, x_ref, wl_ref, bl_ref, wr_ref, g_ref, b_ref,
              out_ref, hbuf, s1, s2):
  p = pl.program_id(0)
  i = pl.program_id(1)

  @pl.when(p == 0)
  def _():
    m = (agg_ref[0] + agg_ref[1]) * _deg_inv(deg_ref)
    h = (jnp.dot(m, wl_ref[...], precision=HI)
         + jnp.dot(x_ref[...], wr_ref[...], precision=HI) + bl_ref[...])
    _bn_stats(i, jnp.maximum(h, 0.0), hbuf, s1, s2)

  @pl.when(p == 1)
  def _():
    h = _bn_apply(i, hbuf, s1, s2, g_ref, b_ref)
    out_ref[0] = h[:, 0:128]
    out_ref[1] = h[:, 128:256]


_tc1 = pl.pallas_call(
    _tc1_body,
    grid=(2, NB),
    in_specs=[
        pl.BlockSpec((2, RB, 128), lambda p, i: (0, i, 0)),
        pl.BlockSpec((2, RB, 128), lambda p, i: (0, i, 0)),
        pl.BlockSpec((RB, 128), lambda p, i: (i, 0)),
        pl.BlockSpec((128, H), lambda p, i: (0, 0)),
        pl.BlockSpec((1, H), lambda p, i: (0, 0)),
        pl.BlockSpec((128, H), lambda p, i: (0, 0)),
        pl.BlockSpec((1, H), lambda p, i: (0, 0)),
        pl.BlockSpec((1, H), lambda p, i: (0, 0)),
    ],
    out_specs=pl.BlockSpec((2, RB, 128), lambda p, i: (0, i, 0)),
    out_shape=jax.ShapeDtypeStruct((2, N, 128), _f32),
    scratch_shapes=[pltpu.VMEM((N, H), _f32), pltpu.VMEM((1, H), _f32),
                    pltpu.VMEM((1, H), _f32)])


def _tc2_body(agg_ref, deg_ref, h1_ref, wl_ref, bl_ref, wr_ref, g_ref, b_ref,
              wp_ref, bp_ref, out_ref, hbuf, s1, s2):
  p = pl.program_id(0)
  i = pl.program_id(1)

  @pl.when(p == 0)
  def _():
    inv = _deg_inv(deg_ref)
    wl = wl_ref[...]
    wr = wr_ref[...]
    h = (jnp.dot(agg_ref[0] * inv, wl[0:128, :], precision=HI)
         + jnp.dot(agg_ref[1] * inv, wl[128:256, :], precision=HI)
         + jnp.dot(h1_ref[0], wr[0:128, :], precision=HI)
         + jnp.dot(h1_ref[1], wr[128:256, :], precision=HI)
         + bl_ref[...])
    _bn_stats(i, jnp.maximum(h, 0.0), hbuf, s1, s2)

  @pl.when(p == 1)
  def _():
    h = _bn_apply(i, hbuf, s1, s2, g_ref, b_ref)
    out_ref[...] = jnp.maximum(jnp.dot(h, wp_ref[...], precision=HI)
                               + bp_ref[...], 0.0)


_tc2 = pl.pallas_call(
    _tc2_body,
    grid=(2, NB),
    in_specs=[
        pl.BlockSpec((2, RB, 128), lambda p, i: (0, i, 0)),
        pl.BlockSpec((2, RB, 128), lambda p, i: (0, i, 0)),
        pl.BlockSpec((2, RB, 128), lambda p, i: (0, i, 0)),
        pl.BlockSpec((H, H), lambda p, i: (0, 0)),
        pl.BlockSpec((1, H), lambda p, i: (0, 0)),
        pl.BlockSpec((H, H), lambda p, i: (0, 0)),
        pl.BlockSpec((1, H), lambda p, i: (0, 0)),
        pl.BlockSpec((1, H), lambda p, i: (0, 0)),
        pl.BlockSpec((H, O), lambda p, i: (0, 0)),
        pl.BlockSpec((1, O), lambda p, i: (0, 0)),
    ],
    out_specs=pl.BlockSpec((RB, O), lambda p, i: (i, 0)),
    out_shape=jax.ShapeDtypeStruct((N, O), _f32),
    scratch_shapes=[pltpu.VMEM((N, H), _f32), pltpu.VMEM((1, H), _f32),
                    pltpu.VMEM((1, H), _f32)])


def kernel(x, edge_index, Wl1, bl1, Wr1, Wl2, bl2, Wr2, g1, b1, g2, b2, Wp,
           bp):
  src = edge_index[0]
  dst = edge_index[1]
  pad = E_PAD - E
  src_p = jnp.concatenate([src, jnp.zeros((pad,), _i32)]).reshape(-1, 128)
  sink = SINK + (jnp.arange(pad, dtype=_i32) % (NPAD - N))
  dst_p = jnp.concatenate([dst, sink]).reshape(-1, 128)

  agg1_all = _make_sc_agg(True)(x, src_p, dst_p)
  degw = _make_sc_deg()(dst_p).reshape(2, N, 128)
  h13 = _tc1(agg1_all.reshape(2, N, 128), degw, x, Wl1,
             bl1.reshape(1, -1), Wr1, g1.reshape(1, -1), b1.reshape(1, -1))
  h1_all = h13.reshape(2 * N, 128)
  agg2_all = _make_sc_agg(False)(h1_all, src_p, dst_p)
  z = _tc2(agg2_all.reshape(2, N, 128), degw, h13, Wl2,
           bl2.reshape(1, -1), Wr2, g2.reshape(1, -1), b2.reshape(1, -1),
           Wp, bp.reshape(1, -1))
  return z


# speedup vs baseline: 6.6266x; 2.1127x over previous
import functools

import jax
import jax.numpy as jnp
from jax import lax
from jax.experimental import pallas as pl
from jax.experimental.pallas import tpu as pltpu, tpu_sc as plsc

N = 10000
E = 320000
D = 128
H = 256
O = 128

NPAD = 10112
SINK = N
E_PAD = 327680
NBLK = E_PAD // 128
HI = lax.Precision.HIGHEST

_f32 = jnp.float32
_i32 = jnp.int32


def _zero_fill(buf, nrows):
  zero16 = jnp.zeros((16,), _f32)
  def zrow(r, _):
    for k in range(8):
      buf[r, pl.ds(k * 16, 16)] = zero16
    return 0
  lax.fori_loop(0, nrows, zrow, 0)


def _zero_shared(shared_ref, zsrc, zb, nrows_chunk):
  for q in range(512 // nrows_chunk):
    pltpu.sync_copy(zsrc, shared_ref.at[pl.ds(zb + q * nrows_chunk,
                                              nrows_chunk)])
  pltpu.sync_copy(zsrc.at[pl.ds(0, 120)], shared_ref.at[pl.ds(zb + 512, 120)])


def _stage_out(shared_ref, hbm_ref, stage_ref, ch, src0, dst0, n):
  for q in range(n // ch):
    pltpu.sync_copy(shared_ref.at[pl.ds(src0 + q * ch, ch)], stage_ref)
    pltpu.sync_copy(stage_ref, hbm_ref.at[pl.ds(dst0 + q * ch, ch)])
  r = n % ch
  if r:
    q = (n // ch) * ch
    pltpu.sync_copy(shared_ref.at[pl.ds(src0 + q, r)],
                    stage_ref.at[pl.ds(0, r)])
    pltpu.sync_copy(stage_ref.at[pl.ds(0, r)],
                    hbm_ref.at[pl.ds(dst0 + q, r)])


def _copy_out(shared_ref, hbm_ref, stage_ref, ch, c, s):
  ob = s * 624
  _stage_out(shared_ref, hbm_ref, stage_ref, ch, ob, c * N + ob, 624)
  @pl.when(s == 0)
  def _():
    _stage_out(shared_ref, hbm_ref, stage_ref, ch, 9984, c * N + 9984, 16)


_MESH = dict(core_axis_name="c", subcore_axis_name="s",
             num_cores=2, num_subcores=16)
IB = 8


@functools.lru_cache(maxsize=None)
def _make_sc_agg(edge_split):
  K = 2
  BT = (NBLK // 32) if edge_split else (NBLK // 16)

  def body(t_all, src2, dst2, out_all, acc, srcv, dstv, rows, semg, sems):
    c = lax.axis_index("c")
    s = lax.axis_index("s")

    _zero_fill(rows, K * 128)
    _zero_shared(acc, rows, s * 632, K * 128)
    plsc.subcore_barrier()

    if edge_split:
      blk0 = (c * 16 + s) * BT
    else:
      blk0 = s * BT
      off = jnp.full((16,), c * N, _i32)

    def rbuf(u):
      return rows.at[pl.ds((u % K) * 128, 128)]

    def chunk(t, _):
      rb = blk0 + t * IB
      pltpu.sync_copy(src2.at[pl.ds(rb, IB)], srcv)
      pltpu.sync_copy(dst2.at[pl.ds(rb, IB)], dstv)
      if not edge_split:
        for j in range(IB):
          for k in range(8):
            srcv[j, pl.ds(k * 16, 16)] = srcv[j, pl.ds(k * 16, 16)] + off
      g = [None] * IB
      sc = [None] * IB
      g[0] = pltpu.async_copy(t_all.at[srcv.at[0]], rbuf(0), semg)
      for u in range(IB):
        g[u].wait()
        sc[u] = pltpu.async_copy(rbuf(u), acc.at[dstv.at[u]], sems, add=True)
        if u + 1 < IB:
          if u >= 1:
            sc[u - 1].wait()
          g[u + 1] = pltpu.async_copy(t_all.at[srcv.at[u + 1]], rbuf(u + 1),
                                      semg)
      sc[IB - 2].wait()
      sc[IB - 1].wait()
      return 0

    lax.fori_loop(0, BT // IB, chunk, 0)
    plsc.subcore_barrier()
    _copy_out(acc, out_all, rows, K * 128, c, s)

  return pl.kernel(
      body,
      out_type=jax.ShapeDtypeStruct((2 * N, 128), _f32),
      mesh=plsc.VectorSubcoreMesh(**_MESH),
      scratch_types=[
          pltpu.VMEM_SHARED((NPAD, 128), _f32),
          pltpu.VMEM((IB, 128), _i32),
          pltpu.VMEM((IB, 128), _i32),
          pltpu.VMEM((K * 128, 128), _f32),
          pltpu.SemaphoreType.DMA,
          pltpu.SemaphoreType.DMA,
      ])


@functools.lru_cache(maxsize=None)
def _make_sc_deg():
  BT = NBLK // 32

  def body(dst2, deg_out, dega, dstv, onesb, sem):
    c = lax.axis_index("c")
    s = lax.axis_index("s")

    _zero_fill(onesb, 128)
    _zero_shared(dega, onesb, s * 632, 128)
    one16 = jnp.ones((16,), _f32)
    def orow(r, _):
      for k in range(8):
        onesb[r, pl.ds(k * 16, 16)] = one16
      return 0
    lax.fori_loop(0, 128, orow, 0)
    plsc.subcore_barrier()

    blk0 = (c * 16 + s) * BT

    def chunk(t, _):
      rb = blk0 + t * IB
      pltpu.sync_copy(dst2.at[pl.ds(rb, IB)], dstv)
      hs = [pltpu.async_copy(onesb, dega.at[dstv.at[u]], sem, add=True)
            for u in range(IB)]
      for h_ in hs:
        h_.wait()
      return 0

    lax.fori_loop(0, BT // IB, chunk, 0)
    plsc.subcore_barrier()
    _copy_out(dega, deg_out, onesb, 128, c, s)

  return pl.kernel(
      body,
      out_type=jax.ShapeDtypeStruct((2 * N, 128), _f32),
      mesh=plsc.VectorSubcoreMesh(**_MESH),
      scratch_types=[
          pltpu.VMEM_SHARED((NPAD, 128), _f32),
          pltpu.VMEM((IB, 128), _i32),
          pltpu.VMEM((128, 128), _f32),
          pltpu.SemaphoreType.DMA,
      ])


NB = 10
RB = N // NB


def _deg_inv(deg_ref):
  d = deg_ref[0][:, 0:1] + deg_ref[1][:, 0:1]
  return 1.0 / jnp.maximum(d, 1.0)


def _bn_stats(i, h, hbuf, s1, s2):
  hbuf[pl.ds(i * RB, RB), :] = h

  @pl.when(i == 0)
  def _():
    s1[...] = jnp.zeros_like(s1)
    s2[...] = jnp.zeros_like(s2)
  s1[...] += jnp.sum(h, axis=0, keepdims=True)
  s2[...] += jnp.sum(h * h, axis=0, keepdims=True)


def _bn_apply(i, hbuf, s1, s2, g_ref, b_ref):
  mu = s1[...] / N
  var = s2[...] / N - mu * mu
  hb = hbuf[pl.ds(i * RB, RB), :]
  return (hb - mu) * lax.rsqrt(var + 1e-5) * g_ref[...] + b_ref[...]


def _tc1_body(agg_ref, deg_ref, x_ref, wl_ref, bl_ref, wr_ref, g_ref, b_ref,
              out_ref, hbuf, s1, s2):
  p = pl.program_id(0)
  i = pl.program_id(1)

  @pl.when(p == 0)
  def _():
    m = (agg_ref[0] + agg_ref[1]) * _deg_inv(deg_ref)
    h = (jnp.dot(m, wl_ref[...], precision=HI)
         + jnp.dot(x_ref[...], wr_ref[...], precision=HI) + bl_ref[...])
    _bn_stats(i, jnp.maximum(h, 0.0), hbuf, s1, s2)

  @pl.when(p == 1)
  def _():
    h = _bn_apply(i, hbuf, s1, s2, g_ref, b_ref)
    out_ref[0] = h[:, 0:128]
    out_ref[1] = h[:, 128:256]


_tc1 = pl.pallas_call(
    _tc1_body,
    grid=(2, NB),
    in_specs=[
        pl.BlockSpec((2, RB, 128), lambda p, i: (0, i, 0)),
        pl.BlockSpec((2, RB, 128), lambda p, i: (0, i, 0)),
        pl.BlockSpec((RB, 128), lambda p, i: (i, 0)),
        pl.BlockSpec((128, H), lambda p, i: (0, 0)),
        pl.BlockSpec((1, H), lambda p, i: (0, 0)),
        pl.BlockSpec((128, H), lambda p, i: (0, 0)),
        pl.BlockSpec((1, H), lambda p, i: (0, 0)),
        pl.BlockSpec((1, H), lambda p, i: (0, 0)),
    ],
    out_specs=pl.BlockSpec((2, RB, 128), lambda p, i: (0, i, 0)),
    out_shape=jax.ShapeDtypeStruct((2, N, 128), _f32),
    scratch_shapes=[pltpu.VMEM((N, H), _f32), pltpu.VMEM((1, H), _f32),
                    pltpu.VMEM((1, H), _f32)])


def _tc2_body(agg_ref, deg_ref, h1_ref, wl_ref, bl_ref, wr_ref, g_ref, b_ref,
              wp_ref, bp_ref, out_ref, hbuf, s1, s2):
  p = pl.program_id(0)
  i = pl.program_id(1)

  @pl.when(p == 0)
  def _():
    inv = _deg_inv(deg_ref)
    wl = wl_ref[...]
    wr = wr_ref[...]
    h = (jnp.dot(agg_ref[0] * inv, wl[0:128, :], precision=HI)
         + jnp.dot(agg_ref[1] * inv, wl[128:256, :], precision=HI)
         + jnp.dot(h1_ref[0], wr[0:128, :], precision=HI)
         + jnp.dot(h1_ref[1], wr[128:256, :], precision=HI)
         + bl_ref[...])
    _bn_stats(i, jnp.maximum(h, 0.0), hbuf, s1, s2)

  @pl.when(p == 1)
  def _():
    h = _bn_apply(i, hbuf, s1, s2, g_ref, b_ref)
    out_ref[...] = jnp.maximum(jnp.dot(h, wp_ref[...], precision=HI)
                               + bp_ref[...], 0.0)


_tc2 = pl.pallas_call(
    _tc2_body,
    grid=(2, NB),
    in_specs=[
        pl.BlockSpec((2, RB, 128), lambda p, i: (0, i, 0)),
        pl.BlockSpec((2, RB, 128), lambda p, i: (0, i, 0)),
        pl.BlockSpec((2, RB, 128), lambda p, i: (0, i, 0)),
        pl.BlockSpec((H, H), lambda p, i: (0, 0)),
        pl.BlockSpec((1, H), lambda p, i: (0, 0)),
        pl.BlockSpec((H, H), lambda p, i: (0, 0)),
        pl.BlockSpec((1, H), lambda p, i: (0, 0)),
        pl.BlockSpec((1, H), lambda p, i: (0, 0)),
        pl.BlockSpec((H, O), lambda p, i: (0, 0)),
        pl.BlockSpec((1, O), lambda p, i: (0, 0)),
    ],
    out_specs=pl.BlockSpec((RB, O), lambda p, i: (i, 0)),
    out_shape=jax.ShapeDtypeStruct((N, O), _f32),
    scratch_shapes=[pltpu.VMEM((N, H), _f32), pltpu.VMEM((1, H), _f32),
                    pltpu.VMEM((1, H), _f32)])


def kernel(x, edge_index, Wl1, bl1, Wr1, Wl2, bl2, Wr2, g1, b1, g2, b2, Wp,
           bp):
  src = edge_index[0]
  dst = edge_index[1]
  pad = E_PAD - E
  spread = jnp.arange(pad, dtype=_i32)
  src_p = jnp.concatenate([src, spread % N]).reshape(-1, 128)
  dst_p = jnp.concatenate([dst, SINK + spread % (NPAD - N)]).reshape(-1, 128)

  agg1_all = _make_sc_agg(True)(x, src_p, dst_p)
  degw = _make_sc_deg()(dst_p).reshape(2, N, 128)
  h13 = _tc1(agg1_all.reshape(2, N, 128), degw, x, Wl1,
             bl1.reshape(1, -1), Wr1, g1.reshape(1, -1), b1.reshape(1, -1))
  h1_all = h13.reshape(2 * N, 128)
  agg2_all = _make_sc_agg(False)(h1_all, src_p, dst_p)
  z = _tc2(agg2_all.reshape(2, N, 128), degw, h13, Wl2,
           bl2.reshape(1, -1), Wr2, g2.reshape(1, -1), b2.reshape(1, -1),
           Wp, bp.reshape(1, -1))
  return z
